# Initial kernel scaffold; baseline (speedup 1.0000x reference)
import jax
import jax.numpy as jnp
from jax.experimental import pallas as pl


def kernel(questions, table):
    raise NotImplementedError("write your pallas kernel here")


# speedup vs baseline: 2.4541x; 2.4541x over previous
import functools

import jax
import jax.numpy as jnp
from jax import lax
from jax.experimental import pallas as pl
from jax.experimental.pallas import tpu as pltpu
from jax.experimental.pallas import tpu_sc as plsc

_D = 64
_L = 50
_NC = 2
_NS = 16
_NW = _NC * _NS
_CB = 16


def _sc_call(questions, table):
    b, l = questions.shape
    d = table.shape[1]
    rows_per_w = b // _NW
    nchunk = rows_per_w // _CB
    mesh = plsc.VectorSubcoreMesh(core_axis_name="c", subcore_axis_name="s")

    @functools.partial(
        pl.kernel,
        out_type=jax.ShapeDtypeStruct((b, d), jnp.float32),
        mesh=mesh,
        scratch_types=[
            pltpu.VMEM((_CB, l), jnp.int32),
            pltpu.VMEM((_CB * l, d), jnp.float32),
            pltpu.VMEM((_CB, d), jnp.float32),
            pltpu.SemaphoreType.DMA,
        ],
        compiler_params=pltpu.CompilerParams(use_tc_tiling_on_sc=False),
    )
    def k(q_hbm, t_hbm, out_hbm, idx_v, rows_v, out_v, sem):
        wid = lax.axis_index("s") * _NC + lax.axis_index("c")
        base_row = wid * rows_per_w

        beta = [
            (lax.iota(jnp.int32, 16).astype(jnp.float32) + (16.0 * kk - 31.0))
            * (1.0 / 800.0)
            for kk in range(4)
        ]

        def chunk_body(ci, carry):
            row0 = base_row + ci * _CB
            pltpu.sync_copy(q_hbm.at[pl.ds(row0, _CB)], idx_v)
            cps = [
                pltpu.async_copy(
                    t_hbm.at[idx_v.at[j]], rows_v.at[pl.ds(j * l, l)], sem
                )
                for j in range(_CB)
            ]
            for cp in cps:
                cp.wait()

            def row_body(r, carry2):
                def l_body(li, accs):
                    alpha = li.astype(jnp.float32) - 24.0
                    new = list(accs)
                    for kk in range(4):
                        v = rows_v[r * l + li, pl.ds(16 * kk, 16)]
                        new[kk] = accs[kk] + v
                        new[4 + kk] = accs[4 + kk] + alpha * v
                    return tuple(new)

                z = jnp.zeros((16,), jnp.float32)
                accs = lax.fori_loop(0, l, l_body, (z,) * 8)
                for kk in range(4):
                    out_v[r, pl.ds(16 * kk, 16)] = (
                        accs[kk] + beta[kk] * accs[4 + kk]
                    )
                return carry2

            lax.fori_loop(0, _CB, row_body, 0)
            pltpu.sync_copy(out_v, out_hbm.at[pl.ds(row0, _CB)])
            return carry

        lax.fori_loop(0, nchunk, chunk_body, 0)

    return k(questions, table)


def kernel(questions, table):
    q = questions.astype(jnp.int32)
    return _sc_call(q, table)


# speedup vs baseline: 2.7769x; 1.1315x over previous
import functools

import jax
import jax.numpy as jnp
from jax import lax
from jax.experimental import pallas as pl
from jax.experimental.pallas import tpu as pltpu
from jax.experimental.pallas import tpu_sc as plsc

_NC = 2
_NS = 16
_NW = _NC * _NS
_CB = 16


def _sc_call(questions, table):
    b, l = questions.shape
    d = table.shape[1]
    nk = d // 16
    rows_per_w = b // _NW
    nchunk = rows_per_w // _CB
    mesh = plsc.VectorSubcoreMesh(core_axis_name="c", subcore_axis_name="s")

    @functools.partial(
        pl.kernel,
        out_type=jax.ShapeDtypeStruct((b, d), jnp.float32),
        mesh=mesh,
        scratch_types=[
            pltpu.VMEM((2, _CB, l), jnp.int32),
            pltpu.VMEM((2, _CB * l, d), jnp.float32),
            pltpu.VMEM((2, _CB, d), jnp.float32),
            pltpu.SemaphoreType.DMA,
            pltpu.SemaphoreType.DMA,
        ],
        compiler_params=pltpu.CompilerParams(use_tc_tiling_on_sc=False),
    )
    def k(q_hbm, t_hbm, out_hbm, idx_v, rows_v, out_v, sem0, sem1):
        wid = lax.axis_index("s") * _NC + lax.axis_index("c")
        base_row = wid * rows_per_w
        sems = [sem0, sem1]

        beta = [
            (lax.iota(jnp.int32, 16).astype(jnp.float32) + (16.0 * kk - 31.0))
            * (1.0 / 800.0)
            for kk in range(nk)
        ]

        def fire(ci, buf):
            row0 = base_row + ci * _CB
            pltpu.sync_copy(q_hbm.at[pl.ds(row0, _CB)], idx_v.at[buf])
            for j in range(_CB):
                pltpu.async_copy(
                    t_hbm.at[idx_v.at[buf].at[j]],
                    rows_v.at[buf].at[pl.ds(j * l, l)],
                    sems[buf],
                )

        def drain(buf):
            pltpu.make_async_copy(
                t_hbm.at[pl.ds(0, _CB * l)], rows_v.at[buf], sems[buf]
            ).wait()

        def compute(ci, buf):
            rows = rows_v.at[buf]
            row0 = base_row + ci * _CB

            def row_body(r, carry):
                base = r * l
                acc0 = [None] * nk
                acc1 = [None] * nk
                for li in range(l):
                    alpha = float(li - 24)
                    for kk in range(nk):
                        v = rows[base + li, pl.ds(16 * kk, 16)]
                        if li == 0:
                            acc0[kk] = v
                            acc1[kk] = alpha * v
                        else:
                            acc0[kk] = acc0[kk] + v
                            if alpha == 1.0:
                                acc1[kk] = acc1[kk] + v
                            elif alpha != 0.0:
                                acc1[kk] = acc1[kk] + alpha * v
                for kk in range(nk):
                    out_v[buf, r, pl.ds(16 * kk, 16)] = (
                        acc0[kk] + beta[kk] * acc1[kk]
                    )
                return carry

            lax.fori_loop(0, _CB, row_body, 0)
            pltpu.sync_copy(out_v.at[buf], out_hbm.at[pl.ds(row0, _CB)])

        fire(0, 0)

        def pair_body(p, carry):
            ci0 = p * 2
            for bb in range(2):
                ci = ci0 + bb
                nxt = ci + 1

                @pl.when(nxt < nchunk)
                def _():
                    fire(nxt, 1 - bb)

                drain(bb)
                compute(ci, bb)
            return carry

        lax.fori_loop(0, nchunk // 2, pair_body, 0)

    return k(questions, table)


def kernel(questions, table):
    q = questions.astype(jnp.int32)
    return _sc_call(q, table)
